# baseline (speedup 1.0000x reference)
import functools
import math

import jax
import jax.numpy as jnp
from jax import lax
from jax.experimental import pallas as pl
from jax.experimental.pallas import tpu as pltpu
from jax.experimental.pallas import tpu_sc as plsc

_D = 64
_SCALE = math.sqrt(_D)
_NC = 2
_NS = 16
_NW = _NC * _NS
_L = 16
_SB = 128
_SP = 133
_NBUF = 4


@functools.lru_cache(maxsize=None)
def _embed_kernel(S, T):
    n_sblk = S // _SB
    assert n_sblk == _NW and T % _NBUF == 0
    n_super = T // _NBUF

    mesh = plsc.VectorSubcoreMesh(core_axis_name="c", subcore_axis_name="s")

    @functools.partial(
        pl.kernel,
        mesh=mesh,
        out_type=jax.ShapeDtypeStruct(
            (T, _D // 8, n_sblk, 8, _SB), jnp.float32
        ),
        scratch_types=[
            pltpu.VMEM((T, _SB), jnp.int32),
            pltpu.VMEM((_NBUF, _SB, _D), jnp.float32),
            pltpu.VMEM((_NBUF, _D // 8, 8, _SP), jnp.float32),
        ]
        + [pltpu.SemaphoreType.DMA] * (2 * _NBUF),
        compiler_params=pltpu.CompilerParams(
            use_tc_tiling_on_sc=False, needs_layout_passes=False
        ),
    )
    def k(idx_hbm, table_hbm, out_hbm, idx_v, rows_v, st_v, *sems):
        sg = sems[:_NBUF]
        ss = sems[_NBUF:]
        wid = lax.axis_index("s") * _NC + lax.axis_index("c")
        s0 = pl.multiple_of(wid * _SB, _SB)
        pltpu.sync_copy(idx_hbm.at[:, pl.ds(s0, _SB)], idx_v)

        def gather_copy(t, b):
            return pltpu.make_async_copy(
                table_hbm.at[idx_v.at[t]],
                rows_v.at[b],
                sg[b],
            )

        def store_copy(t, b):
            return pltpu.make_async_copy(
                st_v.at[b].at[:, :, pl.ds(0, _SB)],
                out_hbm.at[t, :, wid, :, :],
                ss[b],
            )

        gather_copy(0, 0).start()
        gather_copy(1, 1).start()

        iota = lax.iota(jnp.int32, _L)
        fo_vecs = [(2 * g) + (iota // 8) for g in range(_D // _L)]
        fi_vec = lax.rem(iota, 8)

        def super_body(sp, carry):
            for b in range(_NBUF):
                t = _NBUF * sp + b
                bw = (b + 2) % _NBUF

                if b >= 2:
                    store_copy(t - 2, bw).wait()
                else:
                    @pl.when(sp >= 1)
                    def _():
                        store_copy(t - 2, bw).wait()

                if b < 2:
                    gather_copy(t + 2, bw).start()
                else:
                    @pl.when(sp < n_super - 1)
                    def _():
                        gather_copy(t + 2, bw).start()

                gather_copy(t, b).wait()

                rows = rows_v.at[b]
                st = st_v.at[b]

                @plsc.parallel_loop(0, _SB, unroll=4)
                def _(p):
                    pv = jnp.zeros((_L,), jnp.int32) + p
                    for g in range(_D // _L):
                        v = rows[p, pl.ds(g * _L, _L)]
                        plsc.store_scatter(
                            st, [fo_vecs[g], fi_vec, pv], v * _SCALE
                        )

                store_copy(t, b).start()
            return carry

        lax.fori_loop(0, n_super, super_body, 0)
        store_copy(T - 2, (T - 2) % _NBUF).wait()
        store_copy(T - 1, (T - 1) % _NBUF).wait()

    return k


def kernel(inputs, table):
    S, T = inputs.shape
    out5 = _embed_kernel(S, T)(inputs.T, table)
    return out5.transpose(2, 4, 0, 1, 3).reshape(S, T, _D)
